# baseline (speedup 1.0000x reference)
import jax
import jax.numpy as jnp
from jax.experimental import pallas as pl
from jax.experimental.pallas import tpu as pltpu

B, SEQ, D = 32, 196, 768
POOL, LENGTH, TOPK = 100, 10, 5
NSPLIT = 4


def _fused_kernel(x_embed_hbm, x_key_ref, prompt_ref, prompt_key_ref,
                  out_hbm, rs_ref, mean_scratch, copy_sem, mean_sem):
    step = B // NSPLIT
    for i in range(NSPLIT):
        sl = slice(i * step, (i + 1) * step)
        pltpu.make_async_copy(
            x_embed_hbm.at[sl, :],
            out_hbm.at[sl, LENGTH * D:],
            copy_sem,
        ).start()

    xk = x_key_ref[...]
    xk = xk / jnp.maximum(
        jnp.sqrt(jnp.sum(xk * xk, axis=1, keepdims=True)), 1e-12)
    pk = prompt_key_ref[...]
    pk = pk / jnp.maximum(
        jnp.sqrt(jnp.sum(pk * pk, axis=1, keepdims=True)), 1e-12)

    sim = jnp.dot(xk, pk.T, preferred_element_type=jnp.float32)

    m = jnp.max(sim, axis=1, keepdims=True)
    e = jnp.exp(sim - m)
    p = e / jnp.sum(e, axis=1, keepdims=True)

    mean_scratch[...] = jnp.dot(
        p, prompt_ref[...], preferred_element_type=jnp.float32) * (1.0 / POOL)

    pltpu.make_async_copy(
        mean_scratch, out_hbm.at[:, :LENGTH * D], mean_sem).start()

    iota = jax.lax.broadcasted_iota(jnp.int32, (B, POOL), 1)
    v = sim
    total = jnp.float32(0.0)
    for _ in range(TOPK):
        mx = jnp.max(v, axis=1, keepdims=True)
        idx = jnp.min(jnp.where(v >= mx, iota, jnp.int32(POOL)),
                      axis=1, keepdims=True)
        total = total + jnp.sum(mx)
        v = jnp.where(iota == idx, -jnp.inf, v)
    rs_ref[...] = jnp.full((1, 1), total * (1.0 / B), jnp.float32)

    pltpu.make_async_copy(
        mean_scratch, out_hbm.at[:, :LENGTH * D], mean_sem).wait()
    for i in range(NSPLIT):
        sl = slice(i * step, (i + 1) * step)
        pltpu.make_async_copy(
            x_embed_hbm.at[sl, :],
            out_hbm.at[sl, LENGTH * D:],
            copy_sem,
        ).wait()


@jax.jit
def kernel(x_embed, x_key, prompt, prompt_key):
    out_flat, rs = pl.pallas_call(
        _fused_kernel,
        in_specs=[
            pl.BlockSpec(memory_space=pl.ANY),
            pl.BlockSpec(memory_space=pltpu.MemorySpace.VMEM),
            pl.BlockSpec(memory_space=pltpu.MemorySpace.VMEM),
            pl.BlockSpec(memory_space=pltpu.MemorySpace.VMEM),
        ],
        out_specs=[
            pl.BlockSpec(memory_space=pl.ANY),
            pl.BlockSpec(memory_space=pltpu.MemorySpace.VMEM),
        ],
        out_shape=[
            jax.ShapeDtypeStruct((B, (LENGTH + SEQ) * D), jnp.float32),
            jax.ShapeDtypeStruct((1, 1), jnp.float32),
        ],
        scratch_shapes=[
            pltpu.VMEM((B, LENGTH * D), jnp.float32),
            pltpu.SemaphoreType.DMA,
            pltpu.SemaphoreType.DMA,
        ],
    )(x_embed.reshape(B, SEQ * D), x_key,
      prompt.reshape(POOL, LENGTH * D), prompt_key)
    return out_flat.reshape(B, LENGTH + SEQ, D), rs[0, 0]


# speedup vs baseline: 4.5137x; 4.5137x over previous
import jax
import jax.numpy as jnp
from jax.experimental import pallas as pl
from jax.experimental.pallas import tpu as pltpu

B, SEQ, D = 32, 196, 768
POOL, LENGTH, TOPK = 100, 10, 5
W = 1536
NCOPY = SEQ * D // W
NMEAN = LENGTH * D // W


def _fused_kernel(x_ref, x_key_ref, prompt_ref, prompt_key_ref,
                  out_ref, rs_ref):
    j = pl.program_id(0)

    @pl.when(j < NCOPY)
    def _copy():
        out_ref[...] = x_ref[...]

    @pl.when(j >= NCOPY)
    def _mean():
        xk = x_key_ref[...]
        xk = xk / jnp.maximum(
            jnp.sqrt(jnp.sum(xk * xk, axis=1, keepdims=True)), 1e-12)
        pk = prompt_key_ref[...]
        pk = pk / jnp.maximum(
            jnp.sqrt(jnp.sum(pk * pk, axis=1, keepdims=True)), 1e-12)
        sim = jnp.dot(xk, pk.T, preferred_element_type=jnp.float32)

        m = jnp.max(sim, axis=1, keepdims=True)
        e = jnp.exp(sim - m)
        p = e / jnp.sum(e, axis=1, keepdims=True)

        out_ref[...] = jnp.dot(
            p, prompt_ref[...],
            preferred_element_type=jnp.float32) * (1.0 / POOL)

        @pl.when(j == NCOPY)
        def _topk():
            iota = jax.lax.broadcasted_iota(jnp.int32, (B, POOL), 1)
            v = sim
            total = jnp.float32(0.0)
            for _ in range(TOPK):
                mx = jnp.max(v, axis=1, keepdims=True)
                idx = jnp.min(jnp.where(v >= mx, iota, jnp.int32(POOL)),
                              axis=1, keepdims=True)
                total = total + jnp.sum(mx)
                v = jnp.where(iota == idx, -jnp.inf, v)
            rs_ref[...] = jnp.full((1, 1), total * (1.0 / B), jnp.float32)


@jax.jit
def kernel(x_embed, x_key, prompt, prompt_key):
    out_flat, rs = pl.pallas_call(
        _fused_kernel,
        grid=(NCOPY + NMEAN,),
        in_specs=[
            pl.BlockSpec((B, W), lambda j: (0, jnp.minimum(j, NCOPY - 1))),
            pl.BlockSpec((B, 2 * D), lambda j: (0, 0)),
            pl.BlockSpec((POOL, W), lambda j: (0, jnp.maximum(j - NCOPY, 0))),
            pl.BlockSpec((POOL, 2 * D), lambda j: (0, 0)),
        ],
        out_specs=[
            pl.BlockSpec(
                (B, W),
                lambda j: (0, jnp.where(j < NCOPY, j + NMEAN, j - NCOPY))),
            pl.BlockSpec((1, 1), lambda j: (0, 0)),
        ],
        out_shape=[
            jax.ShapeDtypeStruct((B, (LENGTH + SEQ) * D), jnp.float32),
            jax.ShapeDtypeStruct((1, 1), jnp.float32),
        ],
    )(x_embed.reshape(B, SEQ * D), x_key,
      prompt.reshape(POOL, LENGTH * D), prompt_key)
    return out_flat.reshape(B, LENGTH + SEQ, D), rs[0, 0]


# speedup vs baseline: 6.4738x; 1.4343x over previous
import jax
import jax.numpy as jnp
from jax.experimental import pallas as pl
from jax.experimental.pallas import tpu as pltpu

B, SEQ, D = 32, 196, 768
POOL, LENGTH, TOPK = 100, 10, 5
RB = 8
MD = LENGTH * D


def _fused_kernel(x_hbm, x_key_ref, prompt_ref, prompt_key_ref,
                  out_ref, rs_ref, sem):
    j = pl.program_id(0)

    cp = pltpu.make_async_copy(
        x_hbm.at[pl.ds(j * RB, RB), :], out_ref.at[:, MD:], sem)
    cp.start()

    xk = x_key_ref[pl.ds(j * RB, RB), :]
    xk = xk / jnp.maximum(
        jnp.sqrt(jnp.sum(xk * xk, axis=1, keepdims=True)), 1e-12)
    pk = prompt_key_ref[...]
    pk = pk / jnp.maximum(
        jnp.sqrt(jnp.sum(pk * pk, axis=1, keepdims=True)), 1e-12)

    sim = jnp.dot(xk, pk.T, preferred_element_type=jnp.float32)
    m = jnp.max(sim, axis=1, keepdims=True)
    e = jnp.exp(sim - m)
    p = e / jnp.sum(e, axis=1, keepdims=True)

    out_ref[:, :MD] = jnp.dot(
        p, prompt_ref[...], preferred_element_type=jnp.float32) * (1.0 / POOL)

    @pl.when(j == 0)
    def _topk():
        xka = x_key_ref[...]
        xka = xka / jnp.maximum(
            jnp.sqrt(jnp.sum(xka * xka, axis=1, keepdims=True)), 1e-12)
        sima = jnp.dot(xka, pk.T, preferred_element_type=jnp.float32)
        iota = jax.lax.broadcasted_iota(jnp.int32, (B, POOL), 1)
        v = sima
        total = jnp.float32(0.0)
        for _ in range(TOPK):
            mx = jnp.max(v, axis=1, keepdims=True)
            idx = jnp.min(jnp.where(v >= mx, iota, jnp.int32(POOL)),
                          axis=1, keepdims=True)
            total = total + jnp.sum(mx)
            v = jnp.where(iota == idx, -jnp.inf, v)
        rs_ref[...] = jnp.full((1, 1), total * (1.0 / B), jnp.float32)

    cp.wait()


@jax.jit
def kernel(x_embed, x_key, prompt, prompt_key):
    out_flat, rs = pl.pallas_call(
        _fused_kernel,
        grid=(B // RB,),
        in_specs=[
            pl.BlockSpec(memory_space=pl.ANY),
            pl.BlockSpec((B, 2 * D), lambda j: (0, 0)),
            pl.BlockSpec((POOL, MD), lambda j: (0, 0)),
            pl.BlockSpec((POOL, 2 * D), lambda j: (0, 0)),
        ],
        out_specs=[
            pl.BlockSpec((RB, (LENGTH + SEQ) * D), lambda j: (j, 0)),
            pl.BlockSpec((1, 1), lambda j: (0, 0)),
        ],
        out_shape=[
            jax.ShapeDtypeStruct((B, (LENGTH + SEQ) * D), jnp.float32),
            jax.ShapeDtypeStruct((1, 1), jnp.float32),
        ],
        scratch_shapes=[
            pltpu.SemaphoreType.DMA,
        ],
    )(x_embed.reshape(B, SEQ * D), x_key,
      prompt.reshape(POOL, LENGTH * D), prompt_key)
    return out_flat.reshape(B, LENGTH + SEQ, D), rs[0, 0]


# speedup vs baseline: 9.9604x; 1.5386x over previous
import jax
import jax.numpy as jnp
from jax.experimental import pallas as pl
from jax.experimental.pallas import tpu as pltpu

B, SEQ, D = 32, 196, 768
POOL, LENGTH, TOPK = 100, 10, 5
RB = 8


def _fused_kernel(x_ref, x_key_ref, prompt_ref, prompt_key_ref,
                  out_ref, rs_ref):
    j = pl.program_id(0)

    out_ref[:, LENGTH:, :] = x_ref[...]

    xk = x_key_ref[pl.ds(j * RB, RB), :]
    xk = xk / jnp.maximum(
        jnp.sqrt(jnp.sum(xk * xk, axis=1, keepdims=True)), 1e-12)
    pk = prompt_key_ref[...]
    pk = pk / jnp.maximum(
        jnp.sqrt(jnp.sum(pk * pk, axis=1, keepdims=True)), 1e-12)

    sim = jnp.dot(xk, pk.T, preferred_element_type=jnp.float32)
    m = jnp.max(sim, axis=1, keepdims=True)
    e = jnp.exp(sim - m)
    p = e / jnp.sum(e, axis=1, keepdims=True)

    for l in range(LENGTH):
        out_ref[:, l, :] = jnp.dot(
            p, prompt_ref[:, l, :],
            preferred_element_type=jnp.float32) * (1.0 / POOL)

    @pl.when(j == 0)
    def _topk():
        xka = x_key_ref[...]
        xka = xka / jnp.maximum(
            jnp.sqrt(jnp.sum(xka * xka, axis=1, keepdims=True)), 1e-12)
        sima = jnp.dot(xka, pk.T, preferred_element_type=jnp.float32)
        iota = jax.lax.broadcasted_iota(jnp.int32, (B, POOL), 1)
        v = sima
        total = jnp.float32(0.0)
        for _ in range(TOPK):
            mx = jnp.max(v, axis=1, keepdims=True)
            idx = jnp.min(jnp.where(v >= mx, iota, jnp.int32(POOL)),
                          axis=1, keepdims=True)
            total = total + jnp.sum(mx)
            v = jnp.where(iota == idx, -jnp.inf, v)
        rs_ref[...] = jnp.full((1, 1), total * (1.0 / B), jnp.float32)


@jax.jit
def kernel(x_embed, x_key, prompt, prompt_key):
    out, rs = pl.pallas_call(
        _fused_kernel,
        grid=(B // RB,),
        in_specs=[
            pl.BlockSpec((RB, SEQ, D), lambda j: (j, 0, 0)),
            pl.BlockSpec((B, 2 * D), lambda j: (0, 0)),
            pl.BlockSpec((POOL, LENGTH, D), lambda j: (0, 0, 0)),
            pl.BlockSpec((POOL, 2 * D), lambda j: (0, 0)),
        ],
        out_specs=[
            pl.BlockSpec((RB, LENGTH + SEQ, D), lambda j: (j, 0, 0)),
            pl.BlockSpec((1, 1), lambda j: (0, 0)),
        ],
        out_shape=[
            jax.ShapeDtypeStruct((B, LENGTH + SEQ, D), jnp.float32),
            jax.ShapeDtypeStruct((1, 1), jnp.float32),
        ],
    )(x_embed, x_key, prompt, prompt_key)
    return out, rs[0, 0]


# speedup vs baseline: 10.7361x; 1.0779x over previous
import jax
import jax.numpy as jnp
from jax.experimental import pallas as pl
from jax.experimental.pallas import tpu as pltpu

B, SEQ, D = 32, 196, 768
POOL, LENGTH, TOPK = 100, 10, 5
NC = 8
CS = B // NC


def _fused_kernel(x_hbm, x_key_ref, prompt_ref, prompt_key_ref,
                  out_hbm, rs_ref, xbuf, obuf, lsem, ssem):
    for c in range(NC):
        sl = slice(c * CS, (c + 1) * CS)
        pltpu.make_async_copy(
            x_hbm.at[sl, :, :], xbuf.at[sl, :, :], lsem.at[c]).start()

    xk = x_key_ref[...]
    xk = xk / jnp.maximum(
        jnp.sqrt(jnp.sum(xk * xk, axis=1, keepdims=True)), 1e-12)
    pk = prompt_key_ref[...]
    pk = pk / jnp.maximum(
        jnp.sqrt(jnp.sum(pk * pk, axis=1, keepdims=True)), 1e-12)

    sim = jnp.dot(xk, pk.T, preferred_element_type=jnp.float32)
    m = jnp.max(sim, axis=1, keepdims=True)
    e = jnp.exp(sim - m)
    p = e / jnp.sum(e, axis=1, keepdims=True)

    for l in range(LENGTH):
        obuf[:, l, :] = jnp.dot(
            p, prompt_ref[:, l, :],
            preferred_element_type=jnp.float32) * (1.0 / POOL)

    iota = jax.lax.broadcasted_iota(jnp.int32, (B, POOL), 1)
    v = sim
    total = jnp.float32(0.0)
    for _ in range(TOPK):
        mx = jnp.max(v, axis=1, keepdims=True)
        idx = jnp.min(jnp.where(v >= mx, iota, jnp.int32(POOL)),
                      axis=1, keepdims=True)
        total = total + jnp.sum(mx)
        v = jnp.where(iota == idx, -jnp.inf, v)
    rs_ref[...] = jnp.full((1, 1), total * (1.0 / B), jnp.float32)

    for c in range(NC):
        sl = slice(c * CS, (c + 1) * CS)
        pltpu.make_async_copy(
            x_hbm.at[sl, :, :], xbuf.at[sl, :, :], lsem.at[c]).wait()
        obuf[sl, LENGTH:, :] = xbuf[sl, :, :]
        pltpu.make_async_copy(
            obuf.at[sl, :, :], out_hbm.at[sl, :, :], ssem.at[c]).start()

    for c in range(NC):
        sl = slice(c * CS, (c + 1) * CS)
        pltpu.make_async_copy(
            obuf.at[sl, :, :], out_hbm.at[sl, :, :], ssem.at[c]).wait()


@jax.jit
def kernel(x_embed, x_key, prompt, prompt_key):
    out, rs = pl.pallas_call(
        _fused_kernel,
        in_specs=[
            pl.BlockSpec(memory_space=pl.ANY),
            pl.BlockSpec(memory_space=pltpu.MemorySpace.VMEM),
            pl.BlockSpec(memory_space=pltpu.MemorySpace.VMEM),
            pl.BlockSpec(memory_space=pltpu.MemorySpace.VMEM),
        ],
        out_specs=[
            pl.BlockSpec(memory_space=pl.ANY),
            pl.BlockSpec(memory_space=pltpu.MemorySpace.VMEM),
        ],
        out_shape=[
            jax.ShapeDtypeStruct((B, LENGTH + SEQ, D), jnp.float32),
            jax.ShapeDtypeStruct((1, 1), jnp.float32),
        ],
        scratch_shapes=[
            pltpu.VMEM((B, SEQ, D), jnp.float32),
            pltpu.VMEM((B, LENGTH + SEQ, D), jnp.float32),
            pltpu.SemaphoreType.DMA((NC,)),
            pltpu.SemaphoreType.DMA((NC,)),
        ],
    )(x_embed, x_key, prompt, prompt_key)
    return out, rs[0, 0]
